# baseline (speedup 1.0000x reference)
import functools

import jax
import jax.numpy as jnp
from jax import lax
from jax.experimental import pallas as pl
from jax.experimental.pallas import tpu as pltpu
from jax.experimental.pallas import tpu_sc as plsc

N_NODES = 50000
K_EDGE = 16
N_EDGES = 1600000
FILTERS = 256

_J = 1568
_EPAD = _J * 1024
_JB = 16
_GRIDJ = _J // _JB
_BCOL = _JB * 1024

_JH = _J // 2
_EH = _JH * 1024
_C = _JB * 1024
_NCH = _EH // _C
_NPAD = 51200


def _pre_body(ef_ref, sit_ref, ov_ref, od_ref):
    jb = pl.program_id(0)
    colbase = jb * _BCOL
    col2d = colbase + lax.broadcasted_iota(jnp.int32, (K_EDGE, _BCOL), 1)
    v = jnp.where(col2d < N_EDGES, ef_ref[...], 0.0)
    ov_ref[...] = v.reshape(K_EDGE, _JB, 8, 128)
    col1d = colbase + lax.broadcasted_iota(jnp.int32, (1, _BCOL), 1)
    d = jnp.where(col1d < N_EDGES, sit_ref[0:1, :], 0)
    od_ref[...] = d.reshape(_BCOL)


def _pre_pass(edge_features, sparse_indices_t):
    return pl.pallas_call(
        _pre_body,
        grid=(_GRIDJ,),
        in_specs=[
            pl.BlockSpec((K_EDGE, _BCOL), lambda j: (0, j)),
            pl.BlockSpec((2, _BCOL), lambda j: (0, j)),
        ],
        out_specs=[
            pl.BlockSpec((K_EDGE, _JB, 8, 128), lambda j: (0, j, 0, 0)),
            pl.BlockSpec((_BCOL,), lambda j: (j,)),
        ],
        out_shape=[
            jax.ShapeDtypeStruct((K_EDGE, _J, 8, 128), jnp.float32),
            jax.ShapeDtypeStruct((_EPAD,), jnp.int32),
        ],
    )(edge_features, sparse_indices_t)


def _seg_body(dst_hbm, feat_hbm, out_hbm,
              idx0, idx1, val0, val1, acc_v, sem0, sem1):
    c = lax.axis_index("c")
    s = lax.axis_index("s")

    zeros = jnp.zeros((16,), jnp.float32)

    @plsc.parallel_loop(0, N_NODES // 16, unroll=8)
    def zbody(i):
        acc_v[pl.ds(i * 16, 16)] = zeros

    ebase0 = c * _EH
    jbase0 = c * _JH

    fbase0 = s * _EPAD + c * _EH

    def copies(ch, idxb, valb, sem):
        return (
            pltpu.make_async_copy(
                dst_hbm.at[pl.ds(ebase0 + ch * _C, _C)], idxb, sem),
            pltpu.make_async_copy(
                feat_hbm.at[pl.ds(fbase0 + ch * _C, _C)], valb, sem),
        )

    def start(ch, idxb, valb, sem):
        for cp in copies(ch, idxb, valb, sem):
            cp.start()

    def wait(ch, idxb, valb, sem):
        for cp in copies(ch, idxb, valb, sem):
            cp.wait()

    def scatter(idxb, valb):
        @plsc.parallel_loop(0, _C // 128)
        def g_body(t):
            for gg in range(8):
                off = (t * 8 + gg) * 16
                idx = idxb[pl.ds(off, 16)]
                val = valb[pl.ds(off, 16)]
                plsc.addupdate_scatter(acc_v, [idx], val)

    start(0, idx0, val0, sem0)

    def pair_body(p, carry):
        ch0 = 2 * p
        start(ch0 + 1, idx1, val1, sem1)
        wait(ch0, idx0, val0, sem0)
        scatter(idx0, val0)
        start(ch0 + 2, idx0, val0, sem0)
        wait(ch0 + 1, idx1, val1, sem1)
        scatter(idx1, val1)
        return carry

    lax.fori_loop(0, (_NCH - 1) // 2, pair_body, 0)

    wait(_NCH - 1, idx0, val0, sem0)
    scatter(idx0, val0)

    pltpu.sync_copy(acc_v, out_hbm.at[pl.ds((c * K_EDGE + s) * _NPAD, N_NODES)])


_seg_sum = functools.partial(
    pl.kernel,
    out_type=jax.ShapeDtypeStruct((2 * K_EDGE * _NPAD,), jnp.float32),
    mesh=plsc.VectorSubcoreMesh(core_axis_name="c", subcore_axis_name="s"),
    scratch_types=[
        pltpu.VMEM((_C,), jnp.int32),
        pltpu.VMEM((_C,), jnp.int32),
        pltpu.VMEM((_C,), jnp.float32),
        pltpu.VMEM((_C,), jnp.float32),
        pltpu.VMEM((N_NODES,), jnp.float32),
        pltpu.SemaphoreType.DMA,
        pltpu.SemaphoreType.DMA,
    ],
    compiler_params=pltpu.CompilerParams(needs_layout_passes=False),
)(_seg_body)

_BN = 2048


def _mm_body(acc_ref, k_ref, b_ref, o_ref):
    a = acc_ref[0] + acc_ref[1]
    r = lax.dot_general(a, k_ref[...], (((0,), (0,)), ((), ())),
                        preferred_element_type=jnp.float32)
    o_ref[...] = jnp.maximum(r + b_ref[...], 0.0)


def _matmul(acc, kern, bias2d):
    return pl.pallas_call(
        _mm_body,
        grid=(_NPAD // _BN,),
        in_specs=[
            pl.BlockSpec((2, K_EDGE, _BN), lambda i: (0, 0, i)),
            pl.BlockSpec((K_EDGE, FILTERS), lambda i: (0, 0)),
            pl.BlockSpec((1, FILTERS), lambda i: (0, 0)),
        ],
        out_specs=pl.BlockSpec((_BN, FILTERS), lambda i: (i, 0)),
        out_shape=jax.ShapeDtypeStruct((N_NODES, FILTERS), jnp.float32),
    )(acc, kern, bias2d)


def kernel(edge_features, sparse_indices, kernel, bias):
    vals, dst = _pre_pass(edge_features, sparse_indices.astype(jnp.int32).T)
    acc = _seg_sum(dst, vals.reshape(-1))
    acc = acc.reshape(2, K_EDGE, _NPAD)
    return _matmul(acc, kernel, bias.reshape(1, FILTERS))


# speedup vs baseline: 1.0993x; 1.0993x over previous
import functools

import jax
import jax.numpy as jnp
from jax import lax
from jax.experimental import pallas as pl
from jax.experimental.pallas import tpu as pltpu
from jax.experimental.pallas import tpu_sc as plsc

N_NODES = 50000
K_EDGE = 16
N_EDGES = 1600000
FILTERS = 256

_J = 1568
_EPAD = _J * 1024
_JB = 16
_PJB = 32
_GRIDJ = _J // _PJB
_BCOL = _PJB * 1024

_JH = _J // 2
_EH = _JH * 1024
_C = _JB * 1024
_NCH = _EH // _C
_NPAD = 51200


def _pre_body(ef_ref, sit_ref, ov_ref, od_ref):
    jb = pl.program_id(0)
    colbase = jb * _BCOL
    col2d = colbase + lax.broadcasted_iota(jnp.int32, (K_EDGE, _BCOL), 1)
    v = jnp.where(col2d < N_EDGES, ef_ref[...], 0.0)
    ov_ref[...] = v.reshape(K_EDGE, _PJB, 8, 128)
    col1d = colbase + lax.broadcasted_iota(jnp.int32, (1, _BCOL), 1)
    d = jnp.where(col1d < N_EDGES, sit_ref[0:1, :], 0)
    od_ref[...] = d.reshape(_BCOL)


def _pre_pass(edge_features, sparse_indices_t):
    return pl.pallas_call(
        _pre_body,
        grid=(_GRIDJ,),
        in_specs=[
            pl.BlockSpec((K_EDGE, _BCOL), lambda j: (0, j)),
            pl.BlockSpec((2, _BCOL), lambda j: (0, j)),
        ],
        out_specs=[
            pl.BlockSpec((K_EDGE, _PJB, 8, 128), lambda j: (0, j, 0, 0)),
            pl.BlockSpec((_BCOL,), lambda j: (j,)),
        ],
        out_shape=[
            jax.ShapeDtypeStruct((K_EDGE, _J, 8, 128), jnp.float32),
            jax.ShapeDtypeStruct((_EPAD,), jnp.int32),
        ],
    )(edge_features, sparse_indices_t)


def _seg_body(dst_hbm, feat_hbm, out_hbm,
              idx0, idx1, val0, val1, acc_v, sem0, sem1):
    c = lax.axis_index("c")
    s = lax.axis_index("s")

    zeros = jnp.zeros((16,), jnp.float32)

    @plsc.parallel_loop(0, N_NODES // 16, unroll=8)
    def zbody(i):
        acc_v[pl.ds(i * 16, 16)] = zeros

    ebase0 = c * _EH
    jbase0 = c * _JH

    fbase0 = s * _EPAD + c * _EH

    def copies(ch, idxb, valb, sem):
        return (
            pltpu.make_async_copy(
                dst_hbm.at[pl.ds(ebase0 + ch * _C, _C)], idxb, sem),
            pltpu.make_async_copy(
                feat_hbm.at[pl.ds(fbase0 + ch * _C, _C)], valb, sem),
        )

    def start(ch, idxb, valb, sem):
        for cp in copies(ch, idxb, valb, sem):
            cp.start()

    def wait(ch, idxb, valb, sem):
        for cp in copies(ch, idxb, valb, sem):
            cp.wait()

    def scatter(idxb, valb):
        @plsc.parallel_loop(0, _C // 128)
        def g_body(t):
            for gg in range(8):
                off = (t * 8 + gg) * 16
                idx = idxb[pl.ds(off, 16)]
                val = valb[pl.ds(off, 16)]
                plsc.addupdate_scatter(acc_v, [idx], val)

    start(0, idx0, val0, sem0)

    def pair_body(p, carry):
        ch0 = 2 * p
        start(ch0 + 1, idx1, val1, sem1)
        wait(ch0, idx0, val0, sem0)
        scatter(idx0, val0)
        start(ch0 + 2, idx0, val0, sem0)
        wait(ch0 + 1, idx1, val1, sem1)
        scatter(idx1, val1)
        return carry

    lax.fori_loop(0, (_NCH - 1) // 2, pair_body, 0)

    wait(_NCH - 1, idx0, val0, sem0)
    scatter(idx0, val0)

    pltpu.sync_copy(acc_v, out_hbm.at[pl.ds((c * K_EDGE + s) * _NPAD, N_NODES)])


_seg_sum = functools.partial(
    pl.kernel,
    out_type=jax.ShapeDtypeStruct((2 * K_EDGE * _NPAD,), jnp.float32),
    mesh=plsc.VectorSubcoreMesh(core_axis_name="c", subcore_axis_name="s"),
    scratch_types=[
        pltpu.VMEM((_C,), jnp.int32),
        pltpu.VMEM((_C,), jnp.int32),
        pltpu.VMEM((_C,), jnp.float32),
        pltpu.VMEM((_C,), jnp.float32),
        pltpu.VMEM((N_NODES,), jnp.float32),
        pltpu.SemaphoreType.DMA,
        pltpu.SemaphoreType.DMA,
    ],
    compiler_params=pltpu.CompilerParams(needs_layout_passes=False),
)(_seg_body)

_BN = 2048


def _mm_body(acc_ref, k_ref, b_ref, o_ref):
    a = acc_ref[0] + acc_ref[1]
    r = lax.dot_general(a, k_ref[...], (((0,), (0,)), ((), ())),
                        preferred_element_type=jnp.float32)
    o_ref[...] = jnp.maximum(r + b_ref[...], 0.0)


def _matmul(acc, kern, bias2d):
    return pl.pallas_call(
        _mm_body,
        grid=(_NPAD // _BN,),
        in_specs=[
            pl.BlockSpec((2, K_EDGE, _BN), lambda i: (0, 0, i)),
            pl.BlockSpec((K_EDGE, FILTERS), lambda i: (0, 0)),
            pl.BlockSpec((1, FILTERS), lambda i: (0, 0)),
        ],
        out_specs=pl.BlockSpec((_BN, FILTERS), lambda i: (i, 0)),
        out_shape=jax.ShapeDtypeStruct((N_NODES, FILTERS), jnp.float32),
    )(acc, kern, bias2d)


def kernel(edge_features, sparse_indices, kernel, bias):
    vals, dst = _pre_pass(edge_features, sparse_indices.astype(jnp.int32).T)
    acc = _seg_sum(dst, vals.reshape(-1))
    acc = acc.reshape(2, K_EDGE, _NPAD)
    return _matmul(acc, kernel, bias.reshape(1, FILTERS))


# speedup vs baseline: 1.1364x; 1.0337x over previous
import functools

import jax
import jax.numpy as jnp
from jax import lax
from jax.experimental import pallas as pl
from jax.experimental.pallas import tpu as pltpu
from jax.experimental.pallas import tpu_sc as plsc

N_NODES = 50000
K_EDGE = 16
N_EDGES = 1600000
FILTERS = 256

_J = 1568
_EPAD = _J * 1024
_JB = 16
_PJB = 56
_GRIDJ = _J // _PJB
_BCOL = _PJB * 1024

_JH = _J // 2
_EH = _JH * 1024
_C = _JB * 1024
_NCH = _EH // _C
_NPAD = 51200


def _pre_body(ef_ref, sit_ref, ov_ref, od_ref):
    jb = pl.program_id(0)
    colbase = jb * _BCOL
    col2d = colbase + lax.broadcasted_iota(jnp.int32, (K_EDGE, _BCOL), 1)
    v = jnp.where(col2d < N_EDGES, ef_ref[...], 0.0)
    ov_ref[...] = v.reshape(K_EDGE, _PJB, 8, 128)
    col1d = colbase + lax.broadcasted_iota(jnp.int32, (1, _BCOL), 1)
    d = jnp.where(col1d < N_EDGES, sit_ref[0:1, :], 0)
    od_ref[...] = d.reshape(_BCOL)


def _pre_pass(edge_features, sparse_indices_t):
    return pl.pallas_call(
        _pre_body,
        grid=(_GRIDJ,),
        in_specs=[
            pl.BlockSpec((K_EDGE, _BCOL), lambda j: (0, j)),
            pl.BlockSpec((2, _BCOL), lambda j: (0, j)),
        ],
        out_specs=[
            pl.BlockSpec((K_EDGE, _PJB, 8, 128), lambda j: (0, j, 0, 0)),
            pl.BlockSpec((_BCOL,), lambda j: (j,)),
        ],
        out_shape=[
            jax.ShapeDtypeStruct((K_EDGE, _J, 8, 128), jnp.float32),
            jax.ShapeDtypeStruct((_EPAD,), jnp.int32),
        ],
    )(edge_features, sparse_indices_t)


def _seg_body(dst_hbm, feat_hbm, out_hbm,
              idx0, idx1, val0, val1, acc_v, sem0, sem1):
    c = lax.axis_index("c")
    s = lax.axis_index("s")

    zeros = jnp.zeros((16,), jnp.float32)

    @plsc.parallel_loop(0, N_NODES // 16, unroll=8)
    def zbody(i):
        acc_v[pl.ds(i * 16, 16)] = zeros

    ebase0 = c * _EH
    jbase0 = c * _JH

    fbase0 = s * _EPAD + c * _EH

    def copies(ch, idxb, valb, sem):
        return (
            pltpu.make_async_copy(
                dst_hbm.at[pl.ds(ebase0 + ch * _C, _C)], idxb, sem),
            pltpu.make_async_copy(
                feat_hbm.at[pl.ds(fbase0 + ch * _C, _C)], valb, sem),
        )

    def start(ch, idxb, valb, sem):
        for cp in copies(ch, idxb, valb, sem):
            cp.start()

    def wait(ch, idxb, valb, sem):
        for cp in copies(ch, idxb, valb, sem):
            cp.wait()

    def scatter(idxb, valb):
        @plsc.parallel_loop(0, _C // 128)
        def g_body(t):
            for gg in range(8):
                off = (t * 8 + gg) * 16
                idx = idxb[pl.ds(off, 16)]
                val = valb[pl.ds(off, 16)]
                plsc.addupdate_scatter(acc_v, [idx], val)

    start(0, idx0, val0, sem0)

    def pair_body(p, carry):
        ch0 = 2 * p
        start(ch0 + 1, idx1, val1, sem1)
        wait(ch0, idx0, val0, sem0)
        scatter(idx0, val0)
        start(ch0 + 2, idx0, val0, sem0)
        wait(ch0 + 1, idx1, val1, sem1)
        scatter(idx1, val1)
        return carry

    lax.fori_loop(0, (_NCH - 1) // 2, pair_body, 0)

    wait(_NCH - 1, idx0, val0, sem0)
    scatter(idx0, val0)

    pltpu.sync_copy(acc_v, out_hbm.at[pl.ds((c * K_EDGE + s) * _NPAD, N_NODES)])


_seg_sum = functools.partial(
    pl.kernel,
    out_type=jax.ShapeDtypeStruct((2 * K_EDGE * _NPAD,), jnp.float32),
    mesh=plsc.VectorSubcoreMesh(core_axis_name="c", subcore_axis_name="s"),
    scratch_types=[
        pltpu.VMEM((_C,), jnp.int32),
        pltpu.VMEM((_C,), jnp.int32),
        pltpu.VMEM((_C,), jnp.float32),
        pltpu.VMEM((_C,), jnp.float32),
        pltpu.VMEM((N_NODES,), jnp.float32),
        pltpu.SemaphoreType.DMA,
        pltpu.SemaphoreType.DMA,
    ],
    compiler_params=pltpu.CompilerParams(needs_layout_passes=False),
)(_seg_body)

_BN = 2048


def _mm_body(acc_ref, k_ref, b_ref, o_ref):
    a = acc_ref[0] + acc_ref[1]
    r = lax.dot_general(a, k_ref[...], (((0,), (0,)), ((), ())),
                        preferred_element_type=jnp.float32)
    o_ref[...] = jnp.maximum(r + b_ref[...], 0.0)


def _matmul(acc, kern, bias2d):
    return pl.pallas_call(
        _mm_body,
        grid=(_NPAD // _BN,),
        in_specs=[
            pl.BlockSpec((2, K_EDGE, _BN), lambda i: (0, 0, i)),
            pl.BlockSpec((K_EDGE, FILTERS), lambda i: (0, 0)),
            pl.BlockSpec((1, FILTERS), lambda i: (0, 0)),
        ],
        out_specs=pl.BlockSpec((_BN, FILTERS), lambda i: (i, 0)),
        out_shape=jax.ShapeDtypeStruct((N_NODES, FILTERS), jnp.float32),
    )(acc, kern, bias2d)


def kernel(edge_features, sparse_indices, kernel, bias):
    vals, dst = _pre_pass(edge_features, sparse_indices.astype(jnp.int32).T)
    acc = _seg_sum(dst, vals.reshape(-1))
    acc = acc.reshape(2, K_EDGE, _NPAD)
    return _matmul(acc, kernel, bias.reshape(1, FILTERS))


# speedup vs baseline: 1.1478x; 1.0101x over previous
import functools

import jax
import jax.numpy as jnp
from jax import lax
from jax.experimental import pallas as pl
from jax.experimental.pallas import tpu as pltpu
from jax.experimental.pallas import tpu_sc as plsc

N_NODES = 50000
K_EDGE = 16
N_EDGES = 1600000
FILTERS = 256

_J = 1568
_EPAD = _J * 1024
_JB = 16
_PJB = 98
_GRIDJ = _J // _PJB
_BCOL = _PJB * 1024

_JH = _J // 2
_EH = _JH * 1024
_C = _JB * 1024
_NCH = _EH // _C
_NPAD = 51200


def _pre_body(ef_ref, sit_ref, ov_ref, od_ref):
    jb = pl.program_id(0)
    colbase = jb * _BCOL
    col2d = colbase + lax.broadcasted_iota(jnp.int32, (K_EDGE, _BCOL), 1)
    v = jnp.where(col2d < N_EDGES, ef_ref[...], 0.0)
    ov_ref[...] = v.reshape(K_EDGE, _PJB, 8, 128)
    col1d = colbase + lax.broadcasted_iota(jnp.int32, (1, _BCOL), 1)
    d = jnp.where(col1d < N_EDGES, sit_ref[0:1, :], 0)
    od_ref[...] = d.reshape(_BCOL)


def _pre_pass(edge_features, sparse_indices_t):
    return pl.pallas_call(
        _pre_body,
        grid=(_GRIDJ,),
        in_specs=[
            pl.BlockSpec((K_EDGE, _BCOL), lambda j: (0, j)),
            pl.BlockSpec((2, _BCOL), lambda j: (0, j)),
        ],
        out_specs=[
            pl.BlockSpec((K_EDGE, _PJB, 8, 128), lambda j: (0, j, 0, 0)),
            pl.BlockSpec((_BCOL,), lambda j: (j,)),
        ],
        out_shape=[
            jax.ShapeDtypeStruct((K_EDGE, _J, 8, 128), jnp.float32),
            jax.ShapeDtypeStruct((_EPAD,), jnp.int32),
        ],
    )(edge_features, sparse_indices_t)


def _seg_body(dst_hbm, feat_hbm, out_hbm,
              idx0, idx1, val0, val1, acc_v, sem0, sem1):
    c = lax.axis_index("c")
    s = lax.axis_index("s")

    zeros = jnp.zeros((16,), jnp.float32)

    @plsc.parallel_loop(0, N_NODES // 16, unroll=8)
    def zbody(i):
        acc_v[pl.ds(i * 16, 16)] = zeros

    ebase0 = c * _EH
    jbase0 = c * _JH

    fbase0 = s * _EPAD + c * _EH

    def copies(ch, idxb, valb, sem):
        return (
            pltpu.make_async_copy(
                dst_hbm.at[pl.ds(ebase0 + ch * _C, _C)], idxb, sem),
            pltpu.make_async_copy(
                feat_hbm.at[pl.ds(fbase0 + ch * _C, _C)], valb, sem),
        )

    def start(ch, idxb, valb, sem):
        for cp in copies(ch, idxb, valb, sem):
            cp.start()

    def wait(ch, idxb, valb, sem):
        for cp in copies(ch, idxb, valb, sem):
            cp.wait()

    def scatter(idxb, valb):
        @plsc.parallel_loop(0, _C // 128)
        def g_body(t):
            for gg in range(8):
                off = (t * 8 + gg) * 16
                idx = idxb[pl.ds(off, 16)]
                val = valb[pl.ds(off, 16)]
                plsc.addupdate_scatter(acc_v, [idx], val)

    start(0, idx0, val0, sem0)

    def pair_body(p, carry):
        ch0 = 2 * p
        start(ch0 + 1, idx1, val1, sem1)
        wait(ch0, idx0, val0, sem0)
        scatter(idx0, val0)
        start(ch0 + 2, idx0, val0, sem0)
        wait(ch0 + 1, idx1, val1, sem1)
        scatter(idx1, val1)
        return carry

    lax.fori_loop(0, (_NCH - 1) // 2, pair_body, 0)

    wait(_NCH - 1, idx0, val0, sem0)
    scatter(idx0, val0)

    pltpu.sync_copy(acc_v, out_hbm.at[pl.ds((c * K_EDGE + s) * _NPAD, N_NODES)])


_seg_sum = functools.partial(
    pl.kernel,
    out_type=jax.ShapeDtypeStruct((2 * K_EDGE * _NPAD,), jnp.float32),
    mesh=plsc.VectorSubcoreMesh(core_axis_name="c", subcore_axis_name="s"),
    scratch_types=[
        pltpu.VMEM((_C,), jnp.int32),
        pltpu.VMEM((_C,), jnp.int32),
        pltpu.VMEM((_C,), jnp.float32),
        pltpu.VMEM((_C,), jnp.float32),
        pltpu.VMEM((N_NODES,), jnp.float32),
        pltpu.SemaphoreType.DMA,
        pltpu.SemaphoreType.DMA,
    ],
    compiler_params=pltpu.CompilerParams(needs_layout_passes=False),
)(_seg_body)

_BN = 2048


def _mm_body(acc_ref, k_ref, b_ref, o_ref):
    a = acc_ref[0] + acc_ref[1]
    r = lax.dot_general(a, k_ref[...], (((0,), (0,)), ((), ())),
                        preferred_element_type=jnp.float32)
    o_ref[...] = jnp.maximum(r + b_ref[...], 0.0)


def _matmul(acc, kern, bias2d):
    return pl.pallas_call(
        _mm_body,
        grid=(_NPAD // _BN,),
        in_specs=[
            pl.BlockSpec((2, K_EDGE, _BN), lambda i: (0, 0, i)),
            pl.BlockSpec((K_EDGE, FILTERS), lambda i: (0, 0)),
            pl.BlockSpec((1, FILTERS), lambda i: (0, 0)),
        ],
        out_specs=pl.BlockSpec((_BN, FILTERS), lambda i: (i, 0)),
        out_shape=jax.ShapeDtypeStruct((N_NODES, FILTERS), jnp.float32),
    )(acc, kern, bias2d)


def kernel(edge_features, sparse_indices, kernel, bias):
    vals, dst = _pre_pass(edge_features, sparse_indices.astype(jnp.int32).T)
    acc = _seg_sum(dst, vals.reshape(-1))
    acc = acc.reshape(2, K_EDGE, _NPAD)
    return _matmul(acc, kernel, bias.reshape(1, FILTERS))
